# baseline (speedup 1.0000x reference)
import numpy as np
import jax
import jax.numpy as jnp
from jax.experimental import pallas as pl
from jax.experimental.pallas import tpu as pltpu

_EPS = 1e-5
_T = 64
_ROWS = _T * 49

_S0 = np.zeros((36, 16, 9), np.float32)
for ue in range(4):
    for vf in range(4):
        for i in range(3):
            for j in range(3):
                _S0[6 * (ue + i) + (vf + j), ue * 4 + vf, 3 * i + j] = 1.0
_S1 = np.zeros((16, 4, 2, 2), np.float32)
for e in range(2):
    for f in range(2):
        for dr in range(2):
            for dc in range(2):
                _S1[(2 * e + dr) * 4 + (2 * f + dc), e * 2 + f, dr, dc] = 1.0
_APOOL = np.kron(np.eye(_T, dtype=np.float32),
                 np.full((1, 49), 1.0 / 49.0, np.float32))


def _k_stats0(x_ref, w_ref, s_ref, q_ref):
    h = jnp.dot(x_ref[...], w_ref[...], preferred_element_type=jnp.float32)
    s_ref[0, 0, :] = jnp.sum(h, axis=0)
    q_ref[0, 0, :] = jnp.sum(h * h, axis=0)


def _k_stage1(x_ref, w0_ref, sh0_ref, w1a_ref, w1b_ref, h1_ref, s_ref, q_ref):
    h0 = jnp.maximum(
        jnp.dot(x_ref[...], w0_ref[...], preferred_element_type=jnp.float32)
        + sh0_ref[0], 0.0)
    h1a = jnp.dot(h0[:, 0:256], w1a_ref[...],
                  preferred_element_type=jnp.float32)
    h1b = jnp.dot(h0[:, 256:512], w1b_ref[...],
                  preferred_element_type=jnp.float32)
    h1 = jnp.concatenate([h1a, h1b], axis=1)
    h1_ref[...] = h1
    s_ref[0, 0, :] = jnp.sum(h1, axis=0)
    q_ref[0, 0, :] = jnp.sum(h1 * h1, axis=0)


def _k_stage2(h1_ref, sc1_ref, sh1_ref, w2_ref, h2_ref, s_ref, q_ref):
    h1 = jnp.maximum(h1_ref[...] * sc1_ref[0] + sh1_ref[0], 0.0)
    h2 = jnp.dot(h1, w2_ref[...], preferred_element_type=jnp.float32)
    h2_ref[...] = h2
    s_ref[0, 0, :] = jnp.sum(h2, axis=0)
    q_ref[0, 0, :] = jnp.sum(h2 * h2, axis=0)


def _k_stage3(h2_ref, sc2_ref, sh2_ref, ap_ref, wfc_ref, bfc_ref, o_ref):
    h2 = jnp.maximum(h2_ref[...] * sc2_ref[0] + sh2_ref[0], 0.0)
    z = jnp.dot(h2, wfc_ref[...], preferred_element_type=jnp.float32)
    o_ref[...] = jnp.dot(ap_ref[...], z,
                         preferred_element_type=jnp.float32) + bfc_ref[0]


def _scale_shift(s, q, n, g, be):
    mean = s / n
    var = q / n - mean * mean
    scale = g * jax.lax.rsqrt(var + _EPS)
    return scale, be - mean * scale


def kernel(x, W0, b0, g0, be0, W1, b1, g1, be1, W2, b2, g2, be2, Wfc, bfc):
    B = x.shape[0]
    nT = B // _T
    f32 = jnp.float32

    xpad = jnp.pad(x[:, 0], ((0, 0), (1, 1), (1, 1)))
    cols = [jax.lax.slice(xpad, (0, a, b), (B, a + 25, b + 25), (1, 4, 4))
            for a in range(6) for b in range(6)]
    X = jnp.stack(cols, axis=-1).reshape(B * 49, 36)

    w0r = jnp.transpose(W0[:, 0], (1, 2, 0)).reshape(9, 32)
    W0g = jnp.einsum('kpn,nc->kpc', jnp.asarray(_S0), w0r).reshape(36, 512)
    W1g = jnp.einsum('pqde,ocde->pcqo', jnp.asarray(_S1), W1).reshape(512, 256)
    W1a = W1g[0:256, 0:128]
    W1b = W1g[256:512, 128:256]
    W2g = jnp.transpose(W2, (2, 3, 1, 0)).reshape(256, 128)
    wfcT = jnp.transpose(Wfc)
    apool = jnp.asarray(_APOOL)

    cparams = pltpu.CompilerParams(dimension_semantics=("parallel",))

    s0, q0 = pl.pallas_call(
        _k_stats0,
        grid=(nT,),
        in_specs=[
            pl.BlockSpec((_ROWS, 36), lambda i: (i, 0)),
            pl.BlockSpec((36, 512), lambda i: (0, 0)),
        ],
        out_specs=[
            pl.BlockSpec((1, 1, 512), lambda i: (i, 0, 0)),
            pl.BlockSpec((1, 1, 512), lambda i: (i, 0, 0)),
        ],
        out_shape=[
            jax.ShapeDtypeStruct((nT, 1, 512), f32),
            jax.ShapeDtypeStruct((nT, 1, 512), f32),
        ],
        compiler_params=cparams,
    )(X, W0g)
    s0c = jnp.sum(s0, axis=(0, 1)).reshape(16, 32).sum(axis=0)
    q0c = jnp.sum(q0, axis=(0, 1)).reshape(16, 32).sum(axis=0)
    sc0, sh0 = _scale_shift(s0c, q0c, float(B * 28 * 28), g0, be0)
    W0s = W0g * jnp.tile(sc0, 16)[None, :]
    sh0t = jnp.tile(sh0, 16).reshape(1, 512)

    h1p, s1, q1 = pl.pallas_call(
        _k_stage1,
        grid=(nT,),
        in_specs=[
            pl.BlockSpec((_ROWS, 36), lambda i: (i, 0)),
            pl.BlockSpec((36, 512), lambda i: (0, 0)),
            pl.BlockSpec((1, 512), lambda i: (0, 0)),
            pl.BlockSpec((256, 128), lambda i: (0, 0)),
            pl.BlockSpec((256, 128), lambda i: (0, 0)),
        ],
        out_specs=[
            pl.BlockSpec((_ROWS, 256), lambda i: (i, 0)),
            pl.BlockSpec((1, 1, 256), lambda i: (i, 0, 0)),
            pl.BlockSpec((1, 1, 256), lambda i: (i, 0, 0)),
        ],
        out_shape=[
            jax.ShapeDtypeStruct((B * 49, 256), f32),
            jax.ShapeDtypeStruct((nT, 1, 256), f32),
            jax.ShapeDtypeStruct((nT, 1, 256), f32),
        ],
        compiler_params=cparams,
    )(X, W0s, sh0t, W1a, W1b)
    s1c = jnp.sum(s1, axis=(0, 1)).reshape(4, 64).sum(axis=0)
    q1c = jnp.sum(q1, axis=(0, 1)).reshape(4, 64).sum(axis=0)
    sc1, sh1 = _scale_shift(s1c, q1c, float(B * 14 * 14), g1, be1)
    sc1t = jnp.tile(sc1, 4).reshape(1, 256)
    sh1t = jnp.tile(sh1, 4).reshape(1, 256)

    h2p, s2, q2 = pl.pallas_call(
        _k_stage2,
        grid=(nT,),
        in_specs=[
            pl.BlockSpec((_ROWS, 256), lambda i: (i, 0)),
            pl.BlockSpec((1, 256), lambda i: (0, 0)),
            pl.BlockSpec((1, 256), lambda i: (0, 0)),
            pl.BlockSpec((256, 128), lambda i: (0, 0)),
        ],
        out_specs=[
            pl.BlockSpec((_ROWS, 128), lambda i: (i, 0)),
            pl.BlockSpec((1, 1, 128), lambda i: (i, 0, 0)),
            pl.BlockSpec((1, 1, 128), lambda i: (i, 0, 0)),
        ],
        out_shape=[
            jax.ShapeDtypeStruct((B * 49, 128), f32),
            jax.ShapeDtypeStruct((nT, 1, 128), f32),
            jax.ShapeDtypeStruct((nT, 1, 128), f32),
        ],
        compiler_params=cparams,
    )(h1p, sc1t, sh1t, W2g)
    sc2, sh2 = _scale_shift(jnp.sum(s2, axis=(0, 1)), jnp.sum(q2, axis=(0, 1)),
                            float(B * 7 * 7), g2, be2)

    out = pl.pallas_call(
        _k_stage3,
        grid=(nT,),
        in_specs=[
            pl.BlockSpec((_ROWS, 128), lambda i: (i, 0)),
            pl.BlockSpec((1, 128), lambda i: (0, 0)),
            pl.BlockSpec((1, 128), lambda i: (0, 0)),
            pl.BlockSpec((_T, _ROWS), lambda i: (0, 0)),
            pl.BlockSpec((128, 10), lambda i: (0, 0)),
            pl.BlockSpec((1, 10), lambda i: (0, 0)),
        ],
        out_specs=pl.BlockSpec((_T, 10), lambda i: (i, 0)),
        out_shape=jax.ShapeDtypeStruct((B, 10), f32),
        compiler_params=cparams,
    )(h2p, sc2.reshape(1, 128), sh2.reshape(1, 128), apool, wfcT,
      bfc.reshape(1, 10))
    return out
